# Initial kernel scaffold; baseline (speedup 1.0000x reference)
import functools

import jax
import jax.numpy as jnp
from jax.experimental import pallas as pl
from jax.experimental.pallas import tpu as pltpu

N = 10000
NP = 10112


def _tc1_body(x_ref, w_ref, astack_ref, h_ref, al_ref):
    h = jnp.dot(x_ref[...], w_ref[...], preferred_element_type=jnp.float32)
    h_ref[...] = h
    av = jax.lax.dot_general(astack_ref[...], w_ref[...],
                             (((1,), (1,)), ((), ())),
                             preferred_element_type=jnp.float32)
    al_ref[...] = jax.lax.dot_general(av, h, (((1,), (1,)), ((), ())),
                                      preferred_element_type=jnp.float32)


def _tc1(xp, W, astack):
    d_out = W.shape[1]
    return pl.pallas_call(
        _tc1_body,
        in_specs=[pl.BlockSpec((NP, 128), lambda: (0, 0)),
                  pl.BlockSpec((128, d_out), lambda: (0, 0)),
                  pl.BlockSpec((8, 128), lambda: (0, 0))],
        out_specs=[pl.BlockSpec((NP, d_out), lambda: (0, 0)),
                   pl.BlockSpec((8, NP), lambda: (0, 0))],
        out_shape=[jax.ShapeDtypeStruct((NP, d_out), jnp.float32),
                   jax.ShapeDtypeStruct((8, NP), jnp.float32)],
    )(xp, W, astack)


def _tc2_body(acc_ref, dn_ref, b1_ref, w2_ref, av2_ref, h2_ref, al2_ref):
    acc = acc_ref[0] + acc_ref[1]
    dn = dn_ref[0] + dn_ref[1]
    g = acc / (dn + 1e-16) + b1_ref[...]
    h = jnp.where(g > 0, g, jnp.exp(g) - 1.0)
    h2 = jnp.dot(h, w2_ref[...], preferred_element_type=jnp.float32)
    h2_ref[...] = h2
    al2_ref[...] = jax.lax.dot_general(av2_ref[...], h2,
                                       (((1,), (1,)), ((), ())),
                                       preferred_element_type=jnp.float32)


def _tc2(acc, dn, b1, W2, av2):
    d2 = W2.shape[1]
    return pl.pallas_call(
        _tc2_body,
        in_specs=[pl.BlockSpec((2, NP, 128), lambda: (0, 0, 0)),
                  pl.BlockSpec((2, NP, 1), lambda: (0, 0, 0)),
                  pl.BlockSpec((1, 128), lambda: (0, 0)),
                  pl.BlockSpec((128, d2), lambda: (0, 0)),
                  pl.BlockSpec((8, d2), lambda: (0, 0))],
        out_specs=[pl.BlockSpec((NP, d2), lambda: (0, 0)),
                   pl.BlockSpec((8, NP), lambda: (0, 0))],
        out_shape=[jax.ShapeDtypeStruct((NP, d2), jnp.float32),
                   jax.ShapeDtypeStruct((8, NP), jnp.float32)],
    )(acc, dn, b1, W2, av2)


def _tc3_body(acc_ref, dn_ref, b2_ref, out_ref):
    acc = acc_ref[0] + acc_ref[1]
    dn = dn_ref[0] + dn_ref[1]
    out_ref[...] = acc / (dn + 1e-16) + b2_ref[...]


def _tc3(acc, dn, b2):
    d2 = acc.shape[2]
    return pl.pallas_call(
        _tc3_body,
        in_specs=[pl.BlockSpec((2, NP, d2), lambda: (0, 0, 0)),
                  pl.BlockSpec((2, NP, 1), lambda: (0, 0, 0)),
                  pl.BlockSpec((1, d2), lambda: (0, 0))],
        out_specs=pl.BlockSpec((NP, d2), lambda: (0, 0)),
        out_shape=jax.ShapeDtypeStruct((NP, d2), jnp.float32),
    )(acc, dn, b2)


def _edge_phase_jnp(h, asrc, adst, src, dst):
    e = asrc[src] + adst[dst]
    e = jnp.where(e > 0, e, 0.2 * e)
    ex = jnp.exp(e)
    denom = jax.ops.segment_sum(ex, dst, num_segments=NP)
    acc = jax.ops.segment_sum(h[src] * ex[:, None], dst, num_segments=NP)
    z = jnp.zeros_like(acc)
    zd = jnp.zeros_like(denom)
    return jnp.stack([acc, z]), jnp.stack([denom, zd])


def kernel(x, edge_index, W1, a1_src, a1_dst, b1, W2, a2_src, a2_dst, b2):
    loops = jnp.arange(N, dtype=edge_index.dtype)
    src = jnp.concatenate([edge_index[0], loops])
    dst = jnp.concatenate([edge_index[1], loops])

    xp = jnp.zeros((NP, 128), jnp.float32).at[:N].set(x)
    astack1 = jnp.zeros((8, 128), jnp.float32).at[0].set(a1_src).at[1].set(a1_dst)
    av2 = jnp.zeros((8, W2.shape[1]), jnp.float32).at[0].set(a2_src).at[1].set(a2_dst)

    h1, al1 = _tc1(xp, W1, astack1)
    acc1, dn1 = _edge_phase_jnp(h1, al1[0], al1[1], src, dst)
    h2, al2 = _tc2(acc1, dn1[:, :, None], b1.reshape(1, 128), W2, av2)
    acc2, dn2 = _edge_phase_jnp(h2, al2[0], al2[1], src, dst)
    out = _tc3(acc2, dn2[:, :, None], b2.reshape(1, W2.shape[1]))
    return out[:N]


# speedup vs baseline: 1.6699x; 1.6699x over previous
import functools

import jax
import jax.numpy as jnp
from jax.experimental import pallas as pl
from jax.experimental.pallas import tpu as pltpu

N = 10000
NP = 10112


def _tc1_body(x_ref, w_ref, astack_ref, h_ref, al_ref):
    h = jnp.dot(x_ref[...], w_ref[...], preferred_element_type=jnp.float32)
    h_ref[...] = h
    al_ref[...] = jax.lax.dot_general(astack_ref[...], h,
                                      (((1,), (1,)), ((), ())),
                                      preferred_element_type=jnp.float32)


def _tc1(xp, W, astack):
    d_out = W.shape[1]
    return pl.pallas_call(
        _tc1_body,
        in_specs=[pl.BlockSpec((NP, 128), lambda: (0, 0)),
                  pl.BlockSpec((128, d_out), lambda: (0, 0)),
                  pl.BlockSpec((8, 128), lambda: (0, 0))],
        out_specs=[pl.BlockSpec((NP, d_out), lambda: (0, 0)),
                   pl.BlockSpec((8, NP), lambda: (0, 0))],
        out_shape=[jax.ShapeDtypeStruct((NP, d_out), jnp.float32),
                   jax.ShapeDtypeStruct((8, NP), jnp.float32)],
    )(xp, W, astack)


def _tc2_body(acc_ref, dn_ref, b1_ref, w2_ref, av2_ref, h2_ref, al2_ref):
    acc = acc_ref[0] + acc_ref[1]
    dn = dn_ref[0] + dn_ref[1]
    g = acc / (dn + 1e-16) + b1_ref[...]
    h = jnp.where(g > 0, g, jnp.exp(g) - 1.0)
    h2 = jnp.dot(h, w2_ref[...], preferred_element_type=jnp.float32)
    h2_ref[...] = h2
    al2_ref[...] = jax.lax.dot_general(av2_ref[...], h2,
                                       (((1,), (1,)), ((), ())),
                                       preferred_element_type=jnp.float32)


def _tc2(acc, dn, b1, W2, av2):
    d2 = W2.shape[1]
    return pl.pallas_call(
        _tc2_body,
        in_specs=[pl.BlockSpec((2, NP, 128), lambda: (0, 0, 0)),
                  pl.BlockSpec((2, NP, 1), lambda: (0, 0, 0)),
                  pl.BlockSpec((1, 128), lambda: (0, 0)),
                  pl.BlockSpec((128, d2), lambda: (0, 0)),
                  pl.BlockSpec((8, d2), lambda: (0, 0))],
        out_specs=[pl.BlockSpec((NP, d2), lambda: (0, 0)),
                   pl.BlockSpec((8, NP), lambda: (0, 0))],
        out_shape=[jax.ShapeDtypeStruct((NP, d2), jnp.float32),
                   jax.ShapeDtypeStruct((8, NP), jnp.float32)],
    )(acc, dn, b1, W2, av2)


def _tc3_body(acc_ref, dn_ref, b2_ref, out_ref):
    acc = acc_ref[0] + acc_ref[1]
    dn = dn_ref[0] + dn_ref[1]
    out_ref[...] = acc / (dn + 1e-16) + b2_ref[...]


def _tc3(acc, dn, b2):
    d2 = acc.shape[2]
    return pl.pallas_call(
        _tc3_body,
        in_specs=[pl.BlockSpec((2, NP, d2), lambda: (0, 0, 0)),
                  pl.BlockSpec((2, NP, 1), lambda: (0, 0, 0)),
                  pl.BlockSpec((1, d2), lambda: (0, 0))],
        out_specs=pl.BlockSpec((NP, d2), lambda: (0, 0)),
        out_shape=jax.ShapeDtypeStruct((NP, d2), jnp.float32),
    )(acc, dn, b2)


def _edge_phase_jnp(h, asrc, adst, src, dst):
    e = asrc[src] + adst[dst]
    e = jnp.where(e > 0, e, 0.2 * e)
    ex = jnp.exp(e)
    denom = jax.ops.segment_sum(ex, dst, num_segments=NP)
    acc = jax.ops.segment_sum(h[src] * ex[:, None], dst, num_segments=NP)
    z = jnp.zeros_like(acc)
    zd = jnp.zeros_like(denom)
    return jnp.stack([acc, z]), jnp.stack([denom, zd])


def kernel(x, edge_index, W1, a1_src, a1_dst, b1, W2, a2_src, a2_dst, b2):
    loops = jnp.arange(N, dtype=edge_index.dtype)
    src = jnp.concatenate([edge_index[0], loops])
    dst = jnp.concatenate([edge_index[1], loops])

    xp = jnp.zeros((NP, 128), jnp.float32).at[:N].set(x)
    astack1 = jnp.zeros((8, 128), jnp.float32).at[0].set(a1_src).at[1].set(a1_dst)
    av2 = jnp.zeros((8, W2.shape[1]), jnp.float32).at[0].set(a2_src).at[1].set(a2_dst)

    h1, al1 = _tc1(xp, W1, astack1)
    acc1, dn1 = _edge_phase_jnp(h1, al1[0], al1[1], src, dst)
    h2, al2 = _tc2(acc1, dn1[:, :, None], b1.reshape(1, 128), W2, av2)
    acc2, dn2 = _edge_phase_jnp(h2, al2[0], al2[1], src, dst)
    out = _tc3(acc2, dn2[:, :, None], b2.reshape(1, W2.shape[1]))
    return out[:N]


# speedup vs baseline: 20.3243x; 12.1708x over previous
import functools

import jax
import jax.numpy as jnp
from jax import lax
from jax.experimental import pallas as pl
from jax.experimental.pallas import tpu as pltpu
from jax.experimental.pallas import tpu_sc as plsc

N = 10000
NP = 10112
NPT = NP // 16
K = 256
NCH = 41
EC = K * NCH
EP = 32 * EC
D = 128


def _tc1_body(x_ref, w_ref, astack_ref, h_ref, al_ref):
    h = jnp.dot(x_ref[...], w_ref[...], preferred_element_type=jnp.float32)
    h_ref[...] = h
    al_ref[...] = jax.lax.dot_general(astack_ref[...], h,
                                      (((1,), (1,)), ((), ())),
                                      preferred_element_type=jnp.float32)


def _tc1(xp, W, astack):
    return pl.pallas_call(
        _tc1_body,
        in_specs=[pl.BlockSpec((NP, 128), lambda: (0, 0)),
                  pl.BlockSpec((128, 128), lambda: (0, 0)),
                  pl.BlockSpec((8, 128), lambda: (0, 0))],
        out_specs=[pl.BlockSpec((NP, 128), lambda: (0, 0)),
                   pl.BlockSpec((8, NP), lambda: (0, 0))],
        out_shape=[jax.ShapeDtypeStruct((NP, 128), jnp.float32),
                   jax.ShapeDtypeStruct((8, NP), jnp.float32)],
    )(xp, W, astack)


def _tc2_body(acc0_ref, acc1_ref, dn0_ref, dn1_ref, b1_ref, w2_ref, av2_ref,
              h_ref, al2_ref):
    acc = acc0_ref[...] + acc1_ref[...]
    dn = dn0_ref[...] + dn1_ref[...]
    g = acc / (dn + 1e-16) + b1_ref[...]
    h = jnp.where(g > 0, g, jnp.exp(g) - 1.0)
    h_ref[...] = h
    av2w = jax.lax.dot_general(av2_ref[...], w2_ref[...],
                               (((1,), (1,)), ((), ())),
                               preferred_element_type=jnp.float32)
    al2_ref[...] = jax.lax.dot_general(av2w, h, (((1,), (1,)), ((), ())),
                                       preferred_element_type=jnp.float32)


def _tc2(acc0, acc1, dn0, dn1, b1, W2, av2):
    d2 = W2.shape[1]
    return pl.pallas_call(
        _tc2_body,
        in_specs=[pl.BlockSpec((NP, 128), lambda: (0, 0)),
                  pl.BlockSpec((NP, 128), lambda: (0, 0)),
                  pl.BlockSpec((NP, 1), lambda: (0, 0)),
                  pl.BlockSpec((NP, 1), lambda: (0, 0)),
                  pl.BlockSpec((1, 128), lambda: (0, 0)),
                  pl.BlockSpec((128, d2), lambda: (0, 0)),
                  pl.BlockSpec((8, d2), lambda: (0, 0))],
        out_specs=[pl.BlockSpec((NP, 128), lambda: (0, 0)),
                   pl.BlockSpec((8, NP), lambda: (0, 0))],
        out_shape=[jax.ShapeDtypeStruct((NP, 128), jnp.float32),
                   jax.ShapeDtypeStruct((8, NP), jnp.float32)],
    )(acc0, acc1, dn0[:, None], dn1[:, None], b1, W2, av2)


def _tc3_body(acc0_ref, acc1_ref, dn0_ref, dn1_ref, w2_ref, b2_ref, out_ref):
    acc = acc0_ref[...] + acc1_ref[...]
    dn = dn0_ref[...] + dn1_ref[...]
    hagg = acc / (dn + 1e-16)
    out_ref[...] = jnp.dot(hagg, w2_ref[...],
                           preferred_element_type=jnp.float32) + b2_ref[...]


def _tc3(acc0, acc1, dn0, dn1, W2, b2):
    d2 = W2.shape[1]
    return pl.pallas_call(
        _tc3_body,
        in_specs=[pl.BlockSpec((NP, 128), lambda: (0, 0)),
                  pl.BlockSpec((NP, 128), lambda: (0, 0)),
                  pl.BlockSpec((NP, 1), lambda: (0, 0)),
                  pl.BlockSpec((NP, 1), lambda: (0, 0)),
                  pl.BlockSpec((128, d2), lambda: (0, 0)),
                  pl.BlockSpec((1, d2), lambda: (0, 0))],
        out_specs=pl.BlockSpec((NP, d2), lambda: (0, 0)),
        out_shape=jax.ShapeDtypeStruct((NP, d2), jnp.float32),
    )(acc0, acc1, dn0[:, None], dn1[:, None], W2, b2)


def _sc_edge(h, asrc, adst, src1, dst1):
    mesh = plsc.VectorSubcoreMesh(core_axis_name="c", subcore_axis_name="s",
                                  num_cores=2, num_subcores=16)

    @functools.partial(
        pl.kernel, mesh=mesh,
        out_type=[jax.ShapeDtypeStruct((NP, D), jnp.float32),
                  jax.ShapeDtypeStruct((NP, D), jnp.float32),
                  jax.ShapeDtypeStruct((NP,), jnp.float32),
                  jax.ShapeDtypeStruct((NP,), jnp.float32)],
        scratch_types=[
            pltpu.VMEM((K,), jnp.int32),
            pltpu.VMEM((K,), jnp.int32),
            pltpu.VMEM((K,), jnp.float32),
            pltpu.VMEM((K,), jnp.float32),
            pltpu.VMEM((K,), jnp.float32),
            pltpu.VMEM((K, D), jnp.float32),
            pltpu.VMEM((640,), jnp.float32),
            pltpu.VMEM_SHARED((NP, D), jnp.float32),
            pltpu.VMEM_SHARED((NP,), jnp.float32),
            pltpu.SemaphoreType.DMA,
            pltpu.SemaphoreType.DMA,
        ],
    )
    def k(h_hbm, asrc_hbm, adst_hbm, src_hbm, dst_hbm,
          acc0_out, acc1_out, dn0_out, dn1_out,
          idxs_v, idxd_v, sa_v, da_v, ex_v, rows_v, dnb_v,
          acc_sh, dn_sh, sem, sem2):
        cid = lax.axis_index("c")
        sid = lax.axis_index("s")
        slab = cid * 16 + sid
        ebase = slab * EC
        off = pl.multiple_of(sid * NPT, 8)

        zv = jnp.zeros((16,), jnp.float32)

        def zrow_body(i, c):
            for q in range(D // 16):
                rows_v[i, pl.ds(q * 16, 16)] = zv
            return c
        lax.fori_loop(0, K, zrow_body, 0)

        def zden_body(i, c):
            dnb_v[pl.ds(pl.multiple_of(i * 16, 16), 16)] = zv
            return c
        lax.fori_loop(0, 640 // 16, zden_body, 0)

        pltpu.sync_copy(dnb_v.at[pl.ds(0, NPT)], dn_sh.at[pl.ds(off, NPT)])
        pltpu.sync_copy(rows_v, acc_sh.at[pl.ds(off, K)])
        pltpu.sync_copy(rows_v, acc_sh.at[pl.ds(off + K, K)])
        pltpu.sync_copy(rows_v.at[pl.ds(0, NPT - 2 * K)],
                        acc_sh.at[pl.ds(off + 2 * K, NPT - 2 * K)])
        plsc.subcore_barrier()

        def chunk_body(j, carry):
            eoff = pl.multiple_of(ebase + j * K, 8)
            pltpu.sync_copy(src_hbm.at[pl.ds(eoff, K)], idxs_v)
            pltpu.sync_copy(dst_hbm.at[pl.ds(eoff, K)], idxd_v)
            ca = pltpu.async_copy(asrc_hbm.at[idxs_v], sa_v, sem2)
            cb = pltpu.async_copy(adst_hbm.at[idxd_v], da_v, sem2)
            ch = pltpu.async_copy(h_hbm.at[idxs_v], rows_v, sem)
            ca.wait()
            cb.wait()
            for v in range(K // 16):
                sl = pl.ds(v * 16, 16)
                t = sa_v[sl] + da_v[sl]
                e = jnp.where(t > 0, t, 0.2 * t)
                ex_v[sl] = jnp.exp(e)
            pltpu.sync_copy(ex_v, dn_sh.at[idxd_v], add=True)
            ch.wait()

            def group_body(g, c2):
                v16 = ex_v[pl.ds(pl.multiple_of(g * 16, 16), 16)]
                for l in range(16):
                    w = jnp.full((16,), v16[l], jnp.float32)
                    i = g * 16 + l
                    for q in range(D // 16):
                        qs = pl.ds(q * 16, 16)
                        rows_v[i, qs] = rows_v[i, qs] * w
                return c2
            lax.fori_loop(0, K // 16, group_body, 0)

            pltpu.sync_copy(rows_v, acc_sh.at[idxd_v], add=True)
            return carry

        lax.fori_loop(0, NCH, chunk_body, 0)
        plsc.subcore_barrier()

        pltpu.sync_copy(dn_sh.at[pl.ds(off, NPT)], dnb_v.at[pl.ds(0, NPT)])

        @pl.when(cid == 0)
        def _():
            pltpu.sync_copy(dnb_v.at[pl.ds(0, NPT)], dn0_out.at[pl.ds(off, NPT)])

        @pl.when(cid == 1)
        def _():
            pltpu.sync_copy(dnb_v.at[pl.ds(0, NPT)], dn1_out.at[pl.ds(off, NPT)])

        for b, sz in ((0, K), (K, K), (2 * K, NPT - 2 * K)):
            pltpu.sync_copy(acc_sh.at[pl.ds(off + b, sz)],
                            rows_v.at[pl.ds(0, sz)])

            @pl.when(cid == 0)
            def _():
                pltpu.sync_copy(rows_v.at[pl.ds(0, sz)],
                                acc0_out.at[pl.ds(off + b, sz)])

            @pl.when(cid == 1)
            def _():
                pltpu.sync_copy(rows_v.at[pl.ds(0, sz)],
                                acc1_out.at[pl.ds(off + b, sz)])

    return k(h, asrc, adst, src1, dst1)


def kernel(x, edge_index, W1, a1_src, a1_dst, b1, W2, a2_src, a2_dst, b2):
    loops = jnp.arange(N, dtype=edge_index.dtype)
    src = jnp.concatenate([edge_index[0], loops])
    dst = jnp.concatenate([edge_index[1], loops])
    npad = EP - src.shape[0]
    src1 = jnp.pad(src, (0, npad))
    dst1 = jnp.pad(dst, (0, npad), constant_values=N)

    xp = jnp.zeros((NP, 128), jnp.float32).at[:N].set(x)
    astack1 = jnp.zeros((8, 128), jnp.float32).at[0].set(a1_src).at[1].set(a1_dst)
    av2 = jnp.zeros((8, W2.shape[1]), jnp.float32).at[0].set(a2_src).at[1].set(a2_dst)

    h1, al1 = _tc1(xp, W1, astack1)
    a0, a1, d0, d1 = _sc_edge(h1, al1[0], al1[1], src1, dst1)
    h, al2 = _tc2(a0, a1, d0, d1, b1.reshape(1, 128), W2, av2)
    a0, a1, d0, d1 = _sc_edge(h, al2[0], al2[1], src1, dst1)
    out = _tc3(a0, a1, d0, d1, W2, b2.reshape(1, W2.shape[1]))
    return out[:N]


# speedup vs baseline: 24.3902x; 1.2001x over previous
import functools

import jax
import jax.numpy as jnp
from jax import lax
from jax.experimental import pallas as pl
from jax.experimental.pallas import tpu as pltpu
from jax.experimental.pallas import tpu_sc as plsc

N = 10000
NP = 10112
NPT = NP // 16
K = 128
NCH = 82
EC = K * NCH
EP = 32 * EC
D = 128


def _tc1_body(x_ref, w_ref, astack_ref, h_ref, al_ref):
    h = jnp.dot(x_ref[...], w_ref[...], preferred_element_type=jnp.float32)
    h_ref[...] = h
    al_ref[...] = jax.lax.dot_general(astack_ref[...], h,
                                      (((1,), (1,)), ((), ())),
                                      preferred_element_type=jnp.float32)


def _tc1(xp, W, astack):
    return pl.pallas_call(
        _tc1_body,
        in_specs=[pl.BlockSpec((NP, 128), lambda: (0, 0)),
                  pl.BlockSpec((128, 128), lambda: (0, 0)),
                  pl.BlockSpec((8, 128), lambda: (0, 0))],
        out_specs=[pl.BlockSpec((NP, 128), lambda: (0, 0)),
                   pl.BlockSpec((8, NP), lambda: (0, 0))],
        out_shape=[jax.ShapeDtypeStruct((NP, 128), jnp.float32),
                   jax.ShapeDtypeStruct((8, NP), jnp.float32)],
    )(xp, W, astack)


def _tc2_body(acc0_ref, acc1_ref, dn0_ref, dn1_ref, b1_ref, w2_ref, av2_ref,
              h_ref, al2_ref):
    acc = acc0_ref[...] + acc1_ref[...]
    dn = dn0_ref[...] + dn1_ref[...]
    g = acc / (dn + 1e-16) + b1_ref[...]
    h = jnp.where(g > 0, g, jnp.exp(g) - 1.0)
    h_ref[...] = h
    av2w = jax.lax.dot_general(av2_ref[...], w2_ref[...],
                               (((1,), (1,)), ((), ())),
                               preferred_element_type=jnp.float32)
    al2_ref[...] = jax.lax.dot_general(av2w, h, (((1,), (1,)), ((), ())),
                                       preferred_element_type=jnp.float32)


def _tc2(acc0, acc1, dn0, dn1, b1, W2, av2):
    d2 = W2.shape[1]
    return pl.pallas_call(
        _tc2_body,
        in_specs=[pl.BlockSpec((NP, 128), lambda: (0, 0)),
                  pl.BlockSpec((NP, 128), lambda: (0, 0)),
                  pl.BlockSpec((NP, 1), lambda: (0, 0)),
                  pl.BlockSpec((NP, 1), lambda: (0, 0)),
                  pl.BlockSpec((1, 128), lambda: (0, 0)),
                  pl.BlockSpec((128, d2), lambda: (0, 0)),
                  pl.BlockSpec((8, d2), lambda: (0, 0))],
        out_specs=[pl.BlockSpec((NP, 128), lambda: (0, 0)),
                   pl.BlockSpec((8, NP), lambda: (0, 0))],
        out_shape=[jax.ShapeDtypeStruct((NP, 128), jnp.float32),
                   jax.ShapeDtypeStruct((8, NP), jnp.float32)],
    )(acc0, acc1, dn0[:, None], dn1[:, None], b1, W2, av2)


def _tc3_body(acc0_ref, acc1_ref, dn0_ref, dn1_ref, w2_ref, b2_ref, out_ref):
    acc = acc0_ref[...] + acc1_ref[...]
    dn = dn0_ref[...] + dn1_ref[...]
    hagg = acc / (dn + 1e-16)
    out_ref[...] = jnp.dot(hagg, w2_ref[...],
                           preferred_element_type=jnp.float32) + b2_ref[...]


def _tc3(acc0, acc1, dn0, dn1, W2, b2):
    d2 = W2.shape[1]
    return pl.pallas_call(
        _tc3_body,
        in_specs=[pl.BlockSpec((NP, 128), lambda: (0, 0)),
                  pl.BlockSpec((NP, 128), lambda: (0, 0)),
                  pl.BlockSpec((NP, 1), lambda: (0, 0)),
                  pl.BlockSpec((NP, 1), lambda: (0, 0)),
                  pl.BlockSpec((128, d2), lambda: (0, 0)),
                  pl.BlockSpec((1, d2), lambda: (0, 0))],
        out_specs=pl.BlockSpec((NP, d2), lambda: (0, 0)),
        out_shape=jax.ShapeDtypeStruct((NP, d2), jnp.float32),
    )(acc0, acc1, dn0[:, None], dn1[:, None], W2, b2)


def _sc_edge(h, asrc, adst, src1, dst1):
    mesh = plsc.VectorSubcoreMesh(core_axis_name="c", subcore_axis_name="s",
                                  num_cores=2, num_subcores=16)

    @functools.partial(
        pl.kernel, mesh=mesh,
        out_type=[jax.ShapeDtypeStruct((NP, D), jnp.float32),
                  jax.ShapeDtypeStruct((NP, D), jnp.float32),
                  jax.ShapeDtypeStruct((NP,), jnp.float32),
                  jax.ShapeDtypeStruct((NP,), jnp.float32)],
        scratch_types=[
            pltpu.VMEM((K,), jnp.int32),
            pltpu.VMEM((K,), jnp.int32),
            pltpu.VMEM((K,), jnp.int32),
            pltpu.VMEM((K,), jnp.int32),
            pltpu.VMEM((K,), jnp.float32),
            pltpu.VMEM((K,), jnp.float32),
            pltpu.VMEM((K,), jnp.float32),
            pltpu.VMEM((K,), jnp.float32),
            pltpu.VMEM((K,), jnp.float32),
            pltpu.VMEM((K, D), jnp.float32),
            pltpu.VMEM((K, D), jnp.float32),
            pltpu.VMEM((640,), jnp.float32),
            pltpu.VMEM_SHARED((NP, D), jnp.float32),
            pltpu.VMEM_SHARED((NP,), jnp.float32),
            pltpu.SemaphoreType.DMA,
            pltpu.SemaphoreType.DMA,
            pltpu.SemaphoreType.DMA,
            pltpu.SemaphoreType.DMA,
        ],
    )
    def k(h_hbm, asrc_hbm, adst_hbm, src_hbm, dst_hbm,
          acc0_out, acc1_out, dn0_out, dn1_out,
          idxs_a, idxd_a, idxs_b, idxd_b, sa_a, da_a, sa_b, da_b, ex_v,
          rows_a, rows_b, dnb_v, acc_sh, dn_sh, gsem_a, gsem_b,
          ssem_a, ssem_b):
        cid = lax.axis_index("c")
        sid = lax.axis_index("s")
        slab = cid * 16 + sid
        ebase = slab * EC
        off = pl.multiple_of(sid * NPT, 8)

        zv = jnp.zeros((16,), jnp.float32)

        def zrow_body(i, c):
            for q in range(D // 16):
                rows_a[i, pl.ds(q * 16, 16)] = zv
            return c
        lax.fori_loop(0, K, zrow_body, 0)

        def zden_body(i, c):
            dnb_v[pl.ds(pl.multiple_of(i * 16, 16), 16)] = zv
            return c
        lax.fori_loop(0, 640 // 16, zden_body, 0)

        pltpu.sync_copy(dnb_v.at[pl.ds(0, NPT)], dn_sh.at[pl.ds(off, NPT)])
        b = 0
        while b < NPT:
            sz = min(K, NPT - b)
            pltpu.sync_copy(rows_a.at[pl.ds(0, sz)],
                            acc_sh.at[pl.ds(off + b, sz)])
            b += sz
        plsc.subcore_barrier()

        def fetch(j, idxs, idxd, sa, da, rows, gsem, ssem):
            eoff = pl.multiple_of(ebase + j * K, 8)
            pltpu.sync_copy(src_hbm.at[pl.ds(eoff, K)], idxs)
            pltpu.sync_copy(dst_hbm.at[pl.ds(eoff, K)], idxd)
            pltpu.async_copy(asrc_hbm.at[idxs], sa, ssem)
            pltpu.async_copy(adst_hbm.at[idxd], da, ssem)
            pltpu.async_copy(h_hbm.at[idxs], rows, gsem)

        def process(idxs, idxd, sa, da, rows, gsem, ssem):
            pltpu.make_async_copy(asrc_hbm.at[idxs], sa, ssem).wait()
            pltpu.make_async_copy(adst_hbm.at[idxd], da, ssem).wait()
            for v in range(K // 16):
                sl = pl.ds(v * 16, 16)
                t = sa[sl] + da[sl]
                e = jnp.where(t > 0, t, 0.2 * t)
                ex_v[sl] = jnp.exp(e)
            pltpu.sync_copy(ex_v, dn_sh.at[idxd], add=True)
            pltpu.make_async_copy(h_hbm.at[idxs], rows, gsem).wait()

            def group_body(g, c2):
                v16 = ex_v[pl.ds(pl.multiple_of(g * 16, 16), 16)]
                for l in range(16):
                    w = jnp.full((16,), v16[l], jnp.float32)
                    i = g * 16 + l
                    for q in range(D // 16):
                        qs = pl.ds(q * 16, 16)
                        rows[i, qs] = rows[i, qs] * w
                return c2
            lax.fori_loop(0, K // 16, group_body, 0)
            pltpu.sync_copy(rows, acc_sh.at[idxd], add=True)

        bufs_a = (idxs_a, idxd_a, sa_a, da_a, rows_a, gsem_a, ssem_a)
        bufs_b = (idxs_b, idxd_b, sa_b, da_b, rows_b, gsem_b, ssem_b)
        fetch(0, *bufs_a)

        def pair_body(t, carry):
            c0 = t * 2
            fetch(c0 + 1, *bufs_b)
            process(*bufs_a)
            fetch(jnp.minimum(c0 + 2, NCH - 1), *bufs_a)
            process(*bufs_b)
            return carry

        lax.fori_loop(0, NCH // 2, pair_body, 0)
        pltpu.make_async_copy(asrc_hbm.at[idxs_a], sa_a, ssem_a).wait()
        pltpu.make_async_copy(adst_hbm.at[idxd_a], da_a, ssem_a).wait()
        pltpu.make_async_copy(h_hbm.at[idxs_a], rows_a, gsem_a).wait()
        plsc.subcore_barrier()

        pltpu.sync_copy(dn_sh.at[pl.ds(off, NPT)], dnb_v.at[pl.ds(0, NPT)])

        @pl.when(cid == 0)
        def _():
            pltpu.sync_copy(dnb_v.at[pl.ds(0, NPT)], dn0_out.at[pl.ds(off, NPT)])

        @pl.when(cid == 1)
        def _():
            pltpu.sync_copy(dnb_v.at[pl.ds(0, NPT)], dn1_out.at[pl.ds(off, NPT)])

        b = 0
        while b < NPT:
            sz = min(K, NPT - b)
            pltpu.sync_copy(acc_sh.at[pl.ds(off + b, sz)],
                            rows_a.at[pl.ds(0, sz)])

            @pl.when(cid == 0)
            def _():
                pltpu.sync_copy(rows_a.at[pl.ds(0, sz)],
                                acc0_out.at[pl.ds(off + b, sz)])

            @pl.when(cid == 1)
            def _():
                pltpu.sync_copy(rows_a.at[pl.ds(0, sz)],
                                acc1_out.at[pl.ds(off + b, sz)])
            b += sz

    return k(h, asrc, adst, src1, dst1)


def kernel(x, edge_index, W1, a1_src, a1_dst, b1, W2, a2_src, a2_dst, b2):
    loops = jnp.arange(N, dtype=edge_index.dtype)
    src = jnp.concatenate([edge_index[0], loops])
    dst = jnp.concatenate([edge_index[1], loops])
    npad = EP - src.shape[0]
    src1 = jnp.pad(src, (0, npad))
    dst1 = jnp.pad(dst, (0, npad), constant_values=N)

    xp = jnp.zeros((NP, 128), jnp.float32).at[:N].set(x)
    astack1 = jnp.zeros((8, 128), jnp.float32).at[0].set(a1_src).at[1].set(a1_dst)
    av2 = jnp.zeros((8, W2.shape[1]), jnp.float32).at[0].set(a2_src).at[1].set(a2_dst)

    h1, al1 = _tc1(xp, W1, astack1)
    a0, a1, d0, d1 = _sc_edge(h1, al1[0], al1[1], src1, dst1)
    h, al2 = _tc2(a0, a1, d0, d1, b1.reshape(1, 128), W2, av2)
    a0, a1, d0, d1 = _sc_edge(h, al2[0], al2[1], src1, dst1)
    out = _tc3(a0, a1, d0, d1, W2, b2.reshape(1, W2.shape[1]))
    return out[:N]
